# baseline (speedup 1.0000x reference)
import functools

import jax
import jax.numpy as jnp
from jax import lax
from jax.experimental import pallas as pl
from jax.experimental.pallas import tpu as pltpu
from jax.experimental.pallas import tpu_sc as plsc

B = 4096
X_DIM = 3072
Z_DIM = 256
H0 = 2048
H1 = 1024
K = 8192
CODE_DIM = 64

_BM_MLP = 512
_BM_VQ = 512


def _mlp3_kernel(x_ref, w0_ref, b0_ref, w1_ref, b1_ref, w2_ref, b2_ref,
                 out_ref, *, relu_last, cdt):
    dot = functools.partial(jax.lax.dot_general,
                            preferred_element_type=jnp.float32)
    dims = (((1,), (1,)), ((), ()))
    h = jnp.maximum(dot(x_ref[...], w0_ref[...], dims) + b0_ref[...], 0.0)
    h = jnp.maximum(dot(h.astype(cdt), w1_ref[...], dims) + b1_ref[...], 0.0)
    o = dot(h.astype(cdt), w2_ref[...], dims) + b2_ref[...]
    if relu_last:
        o = jnp.maximum(o, 0.0)
    out_ref[...] = o


def _mlp3(x, w0, b0, w1, b1, w2, b2, *, bm, relu_last=False,
          cdt=jnp.float32):
    m = x.shape[0]
    d_in = x.shape[1]
    d0, d1, d2 = w0.shape[0], w1.shape[0], w2.shape[0]
    grid = (m // bm,)
    full = lambda shape: pl.BlockSpec(shape, lambda i: (0, 0))
    return pl.pallas_call(
        functools.partial(_mlp3_kernel, relu_last=relu_last, cdt=cdt),
        grid=grid,
        in_specs=[
            pl.BlockSpec((bm, d_in), lambda i: (i, 0)),
            full((d0, d_in)), full((1, d0)),
            full((d1, d0)), full((1, d1)),
            full((d2, d1)), full((1, d2)),
        ],
        out_specs=pl.BlockSpec((bm, d2), lambda i: (i, 0)),
        out_shape=jax.ShapeDtypeStruct((m, d2), jnp.float32),
        compiler_params=pltpu.CompilerParams(
            dimension_semantics=("parallel",)),
    )(x.astype(cdt), w0.astype(cdt), b0.reshape(1, -1),
      w1.astype(cdt), b1.reshape(1, -1), w2.astype(cdt), b2.reshape(1, -1))


def _vq_kernel(z_ref, cb_ref, idx_ref):
    z = z_ref[...]
    cb = cb_ref[...]
    dot = jax.lax.dot_general(
        z, cb, (((1,), (1,)), ((), ())),
        preferred_element_type=jnp.float32)
    zn = jnp.sum(z * z, axis=1, keepdims=True)
    cn = jnp.sum(cb * cb, axis=1)[None, :]
    d = (zn + cn) - 2.0 * dot
    m = jnp.min(d, axis=1, keepdims=True)
    iota = jax.lax.broadcasted_iota(jnp.int32, d.shape, 1)
    idx_ref[...] = jnp.min(jnp.where(d == m, iota, K),
                           axis=1, keepdims=True)


def _vq_argmin(z_e, codebook, *, bm):
    m = z_e.shape[0]
    return pl.pallas_call(
        _vq_kernel,
        grid=(m // bm,),
        in_specs=[
            pl.BlockSpec((bm, CODE_DIM), lambda i: (i, 0)),
            pl.BlockSpec((K, CODE_DIM), lambda i: (0, 0)),
        ],
        out_specs=pl.BlockSpec((bm, 1), lambda i: (i, 0)),
        out_shape=jax.ShapeDtypeStruct((m, 1), jnp.int32),
        compiler_params=pltpu.CompilerParams(
            dimension_semantics=("parallel",)),
    )(z_e, codebook)


_GCHUNK = 128


def _sc_gather(table, idx):
    n = idx.shape[0]
    d = table.shape[1]
    info = plsc.get_sparse_core_info()
    nw = info.num_cores * info.num_subcores
    b_per_w = n // nw
    nchunk = b_per_w // _GCHUNK
    mesh = plsc.VectorSubcoreMesh(core_axis_name="c", subcore_axis_name="s")

    @functools.partial(
        pl.kernel, mesh=mesh,
        out_type=jax.ShapeDtypeStruct((n, d), jnp.float32),
        scratch_types=[
            pltpu.VMEM((_GCHUNK,), jnp.int32),
            pltpu.VMEM((_GCHUNK, d), jnp.float32),
            pltpu.SemaphoreType.DMA,
        ],
    )
    def k(table_hbm, idx_hbm, out_hbm, idx_v, rows_v, sem):
        wid = lax.axis_index("s") * info.num_cores + lax.axis_index("c")
        for c in range(nchunk):
            base = wid * b_per_w + c * _GCHUNK
            pltpu.sync_copy(idx_hbm.at[pl.ds(base, _GCHUNK)], idx_v)
            pltpu.async_copy(table_hbm.at[idx_v], rows_v, sem).wait()
            pltpu.sync_copy(rows_v, out_hbm.at[pl.ds(base, _GCHUNK)])

    return k(table, idx)


def kernel(x, enc0_w, enc0_b, enc1_w, enc1_b, z_w, z_b,
           dec0_w, dec0_b, dec1_w, dec1_b, out_w, out_b, codebook):
    bm = min(_BM_MLP, x.shape[0])
    z = _mlp3(x, enc0_w, enc0_b, enc1_w, enc1_b, z_w, z_b, bm=bm)
    z_e = z.reshape(-1, CODE_DIM)
    idx = _vq_argmin(z_e, codebook, bm=min(_BM_VQ, z_e.shape[0]))
    cb_pad = jnp.pad(codebook, ((0, 0), (0, 128 - CODE_DIM)))
    z_q = _sc_gather(cb_pad, idx.reshape(-1))[:, :CODE_DIM]
    net = z_q.reshape(-1, Z_DIM)
    return _mlp3(net, dec0_w, dec0_b, dec1_w, dec1_b, out_w, out_b, bm=bm,
                 cdt=jnp.bfloat16)


# speedup vs baseline: 1.0252x; 1.0252x over previous
import functools

import jax
import jax.numpy as jnp
from jax import lax
from jax.experimental import pallas as pl
from jax.experimental.pallas import tpu as pltpu
from jax.experimental.pallas import tpu_sc as plsc

B = 4096
X_DIM = 3072
Z_DIM = 256
H0 = 2048
H1 = 1024
K = 8192
CODE_DIM = 64

_BM_MLP = 512
_BM_VQ = 512


def _mlp3_kernel(x_ref, w0_ref, b0_ref, w1_ref, b1_ref, w2_ref, b2_ref,
                 out_ref, *, relu_last, cdt):
    dot = functools.partial(jax.lax.dot_general,
                            preferred_element_type=jnp.float32)
    dims = (((1,), (1,)), ((), ()))
    h = jnp.maximum(dot(x_ref[...], w0_ref[...], dims) + b0_ref[...], 0.0)
    h = jnp.maximum(dot(h.astype(cdt), w1_ref[...], dims) + b1_ref[...], 0.0)
    o = dot(h.astype(cdt), w2_ref[...], dims) + b2_ref[...]
    if relu_last:
        o = jnp.maximum(o, 0.0)
    out_ref[...] = o


def _mlp3(x, w0, b0, w1, b1, w2, b2, *, bm, relu_last=False,
          cdt=jnp.float32):
    m = x.shape[0]
    d_in = x.shape[1]
    d0, d1, d2 = w0.shape[0], w1.shape[0], w2.shape[0]
    grid = (m // bm,)
    full = lambda shape: pl.BlockSpec(shape, lambda i: (0, 0))
    return pl.pallas_call(
        functools.partial(_mlp3_kernel, relu_last=relu_last, cdt=cdt),
        grid=grid,
        in_specs=[
            pl.BlockSpec((bm, d_in), lambda i: (i, 0)),
            full((d0, d_in)), full((1, d0)),
            full((d1, d0)), full((1, d1)),
            full((d2, d1)), full((1, d2)),
        ],
        out_specs=pl.BlockSpec((bm, d2), lambda i: (i, 0)),
        out_shape=jax.ShapeDtypeStruct((m, d2), jnp.float32),
        compiler_params=pltpu.CompilerParams(
            dimension_semantics=("parallel",)),
    )(x.astype(cdt), w0.astype(cdt), b0.reshape(1, -1),
      w1.astype(cdt), b1.reshape(1, -1), w2.astype(cdt), b2.reshape(1, -1))


def _vq_kernel(z_ref, cb_ref, idx_ref):
    z = z_ref[...]
    cb = cb_ref[...]
    dot = jax.lax.dot_general(
        z, cb, (((1,), (1,)), ((), ())),
        preferred_element_type=jnp.float32)
    zn = jnp.sum(z * z, axis=1, keepdims=True)
    cn = jnp.sum(cb * cb, axis=1)[None, :]
    d = (zn + cn) - 2.0 * dot
    m = jnp.min(d, axis=1, keepdims=True)
    iota = jax.lax.broadcasted_iota(jnp.int32, d.shape, 1)
    idx_ref[...] = jnp.min(jnp.where(d == m, iota, K),
                           axis=1, keepdims=True)


def _vq_argmin(z_e, codebook, *, bm):
    m = z_e.shape[0]
    return pl.pallas_call(
        _vq_kernel,
        grid=(m // bm,),
        in_specs=[
            pl.BlockSpec((bm, CODE_DIM), lambda i: (i, 0)),
            pl.BlockSpec((K, CODE_DIM), lambda i: (0, 0)),
        ],
        out_specs=pl.BlockSpec((bm, 1), lambda i: (i, 0)),
        out_shape=jax.ShapeDtypeStruct((m, 1), jnp.int32),
        compiler_params=pltpu.CompilerParams(
            dimension_semantics=("parallel",)),
    )(z_e, codebook)


_GCHUNK = 128


def _sc_gather(table, idx):
    n = idx.shape[0]
    d = table.shape[1]
    info = plsc.get_sparse_core_info()
    nw = info.num_cores * info.num_subcores
    b_per_w = n // nw
    nchunk = b_per_w // _GCHUNK
    mesh = plsc.VectorSubcoreMesh(core_axis_name="c", subcore_axis_name="s")

    @functools.partial(
        pl.kernel, mesh=mesh,
        out_type=jax.ShapeDtypeStruct((n, d), jnp.float32),
        scratch_types=[
            pltpu.VMEM((_GCHUNK,), jnp.int32),
            pltpu.VMEM((_GCHUNK, d), jnp.float32),
            pltpu.SemaphoreType.DMA,
        ],
    )
    def k(table_hbm, idx_hbm, out_hbm, idx_v, rows_v, sem):
        wid = lax.axis_index("s") * info.num_cores + lax.axis_index("c")
        for c in range(nchunk):
            base = wid * b_per_w + c * _GCHUNK
            pltpu.sync_copy(idx_hbm.at[pl.ds(base, _GCHUNK)], idx_v)
            pltpu.async_copy(table_hbm.at[idx_v], rows_v, sem).wait()
            pltpu.sync_copy(rows_v, out_hbm.at[pl.ds(base, _GCHUNK)])

    return k(table, idx)


def kernel(x, enc0_w, enc0_b, enc1_w, enc1_b, z_w, z_b,
           dec0_w, dec0_b, dec1_w, dec1_b, out_w, out_b, codebook):
    bm = min(_BM_MLP, x.shape[0])
    z = _mlp3(x, enc0_w, enc0_b, enc1_w, enc1_b, z_w, z_b, bm=bm)
    z_e = z.reshape(-1, CODE_DIM)
    idx = _vq_argmin(z_e, codebook, bm=min(_BM_VQ, z_e.shape[0]))
    cb_pad = jnp.pad(codebook, ((0, 0), (0, 128 - CODE_DIM)))
    z_q = _sc_gather(cb_pad, idx.reshape(-1))[:, :CODE_DIM]
    net = z_q.reshape(-1, Z_DIM)
    return _mlp3(net, dec0_w, dec0_b, dec1_w, dec1_b, out_w, out_b, bm=bm)


# speedup vs baseline: 1.1017x; 1.0746x over previous
import functools

import jax
import jax.numpy as jnp
from jax import lax
from jax.experimental import pallas as pl
from jax.experimental.pallas import tpu as pltpu
from jax.experimental.pallas import tpu_sc as plsc

B = 4096
X_DIM = 3072
Z_DIM = 256
H0 = 2048
H1 = 1024
K = 8192
CODE_DIM = 64

_BM_MLP = 512
_BM_VQ = 512


def _mlp3_kernel(x_ref, w0_ref, b0_ref, w1_ref, b1_ref, w2_ref, b2_ref,
                 out_ref, *, relu_last, cdt):
    dot = functools.partial(jax.lax.dot_general,
                            preferred_element_type=jnp.float32)
    dims = (((1,), (1,)), ((), ()))
    h = jnp.maximum(dot(x_ref[...], w0_ref[...], dims) + b0_ref[...], 0.0)
    h = jnp.maximum(dot(h.astype(cdt), w1_ref[...], dims) + b1_ref[...], 0.0)
    o = dot(h.astype(cdt), w2_ref[...], dims) + b2_ref[...]
    if relu_last:
        o = jnp.maximum(o, 0.0)
    out_ref[...] = o


def _mlp3(x, w0, b0, w1, b1, w2, b2, *, bm, relu_last=False,
          cdt=jnp.float32):
    m = x.shape[0]
    d_in = x.shape[1]
    d0, d1, d2 = w0.shape[0], w1.shape[0], w2.shape[0]
    grid = (m // bm,)
    full = lambda shape: pl.BlockSpec(shape, lambda i: (0, 0))
    return pl.pallas_call(
        functools.partial(_mlp3_kernel, relu_last=relu_last, cdt=cdt),
        grid=grid,
        in_specs=[
            pl.BlockSpec((bm, d_in), lambda i: (i, 0)),
            full((d0, d_in)), full((1, d0)),
            full((d1, d0)), full((1, d1)),
            full((d2, d1)), full((1, d2)),
        ],
        out_specs=pl.BlockSpec((bm, d2), lambda i: (i, 0)),
        out_shape=jax.ShapeDtypeStruct((m, d2), jnp.float32),
        compiler_params=pltpu.CompilerParams(
            dimension_semantics=("parallel",)),
    )(x.astype(cdt), w0.astype(cdt), b0.reshape(1, -1),
      w1.astype(cdt), b1.reshape(1, -1), w2.astype(cdt), b2.reshape(1, -1))


def _vq_kernel(z_ref, cb_ref, idx_ref):
    z = z_ref[...]
    cb = cb_ref[...]
    dot = jax.lax.dot_general(
        z, cb, (((1,), (1,)), ((), ())),
        preferred_element_type=jnp.float32)
    zn = jnp.sum(z * z, axis=1, keepdims=True)
    cn = jnp.sum(cb * cb, axis=1)[None, :]
    d = (zn + cn) - 2.0 * dot
    idx_ref[...] = jnp.argmin(d, axis=1).astype(jnp.int32)[:, None]


def _vq_argmin(z_e, codebook, *, bm):
    m = z_e.shape[0]
    return pl.pallas_call(
        _vq_kernel,
        grid=(m // bm,),
        in_specs=[
            pl.BlockSpec((bm, CODE_DIM), lambda i: (i, 0)),
            pl.BlockSpec((K, CODE_DIM), lambda i: (0, 0)),
        ],
        out_specs=pl.BlockSpec((bm, 1), lambda i: (i, 0)),
        out_shape=jax.ShapeDtypeStruct((m, 1), jnp.int32),
        compiler_params=pltpu.CompilerParams(
            dimension_semantics=("parallel",)),
    )(z_e, codebook)


_GCHUNK = 128


def _sc_gather(table, idx):
    n = idx.shape[0]
    d = table.shape[1]
    info = plsc.get_sparse_core_info()
    nw = info.num_cores * info.num_subcores
    b_per_w = n // nw
    nchunk = b_per_w // _GCHUNK
    mesh = plsc.VectorSubcoreMesh(core_axis_name="c", subcore_axis_name="s")

    @functools.partial(
        pl.kernel, mesh=mesh,
        out_type=jax.ShapeDtypeStruct((n, d), jnp.float32),
        scratch_types=[
            pltpu.VMEM((_GCHUNK,), jnp.int32),
            pltpu.VMEM((_GCHUNK, d), jnp.float32),
            pltpu.SemaphoreType.DMA,
        ],
    )
    def k(table_hbm, idx_hbm, out_hbm, idx_v, rows_v, sem):
        wid = lax.axis_index("s") * info.num_cores + lax.axis_index("c")
        for c in range(nchunk):
            base = wid * b_per_w + c * _GCHUNK
            pltpu.sync_copy(idx_hbm.at[pl.ds(base, _GCHUNK)], idx_v)
            pltpu.async_copy(table_hbm.at[idx_v], rows_v, sem).wait()
            pltpu.sync_copy(rows_v, out_hbm.at[pl.ds(base, _GCHUNK)])

    return k(table, idx)


def kernel(x, enc0_w, enc0_b, enc1_w, enc1_b, z_w, z_b,
           dec0_w, dec0_b, dec1_w, dec1_b, out_w, out_b, codebook):
    bm = min(_BM_MLP, x.shape[0])
    z = _mlp3(x, enc0_w, enc0_b, enc1_w, enc1_b, z_w, z_b, bm=bm)
    z_e = z.reshape(-1, CODE_DIM)
    idx = _vq_argmin(z_e, codebook, bm=min(_BM_VQ, z_e.shape[0]))
    cb_pad = jnp.pad(codebook, ((0, 0), (0, 128 - CODE_DIM)))
    z_q_pad = _sc_gather(cb_pad, idx.reshape(-1))
    groups = Z_DIM // CODE_DIM
    net = z_q_pad.reshape(-1, groups * 128)
    dec0_w_pad = jnp.pad(dec0_w.reshape(dec0_w.shape[0], groups, CODE_DIM),
                         ((0, 0), (0, 0), (0, 128 - CODE_DIM)))
    dec0_w_pad = dec0_w_pad.reshape(dec0_w.shape[0], groups * 128)
    return _mlp3(net, dec0_w_pad, dec0_b, dec1_w, dec1_b, out_w, out_b, bm=bm)
